# baseline (speedup 1.0000x reference)
import jax
import jax.numpy as jnp
from jax.experimental import pallas as pl
from jax.experimental.pallas import tpu as pltpu

_NSLICES = 8


def _copy_tables(ent_ref, rel_ref, ent_out, rel_out, *sems):
    n = ent_ref.shape[0]
    chunk = n // _NSLICES
    copies = []
    for i in range(_NSLICES):
        cp = pltpu.make_async_copy(
            ent_ref.at[pl.ds(i * chunk, chunk)],
            ent_out.at[pl.ds(i * chunk, chunk)],
            sems[i],
        )
        cp.start()
        copies.append(cp)
    rcp = pltpu.make_async_copy(rel_ref, rel_out, sems[_NSLICES])
    rcp.start()
    for cp in copies:
        cp.wait()
    rcp.wait()


def kernel(x_dict, edge_index, entity_emb, rel_emb):
    del x_dict, edge_index
    ent, rel = pl.pallas_call(
        _copy_tables,
        in_specs=[
            pl.BlockSpec(memory_space=pltpu.MemorySpace.HBM),
            pl.BlockSpec(memory_space=pltpu.MemorySpace.HBM),
        ],
        out_specs=[
            pl.BlockSpec(memory_space=pltpu.MemorySpace.HBM),
            pl.BlockSpec(memory_space=pltpu.MemorySpace.HBM),
        ],
        out_shape=[
            jax.ShapeDtypeStruct(entity_emb.shape, entity_emb.dtype),
            jax.ShapeDtypeStruct(rel_emb.shape, rel_emb.dtype),
        ],
        scratch_shapes=[pltpu.SemaphoreType.DMA] * (_NSLICES + 1),
    )(entity_emb, rel_emb)
    return (ent, rel)


# speedup vs baseline: 48.4521x; 48.4521x over previous
import jax
import jax.numpy as jnp
from jax.experimental import pallas as pl
from jax.experimental.pallas import tpu as pltpu

_BLK = 25000


def _copy_tables(ent_ref, rel_ref, ent_out, rel_out):
    ent_out[...] = ent_ref[...]

    @pl.when(pl.program_id(0) == 0)
    def _():
        rel_out[...] = rel_ref[...]


def kernel(x_dict, edge_index, entity_emb, rel_emb):
    del x_dict, edge_index
    n_ent, d_ent = entity_emb.shape
    n_rel, d_rel = rel_emb.shape
    ent, rel = pl.pallas_call(
        _copy_tables,
        grid=(n_ent // _BLK,),
        in_specs=[
            pl.BlockSpec((_BLK, d_ent), lambda i: (i, 0)),
            pl.BlockSpec((n_rel, d_rel), lambda i: (0, 0)),
        ],
        out_specs=[
            pl.BlockSpec((_BLK, d_ent), lambda i: (i, 0)),
            pl.BlockSpec((n_rel, d_rel), lambda i: (0, 0)),
        ],
        out_shape=[
            jax.ShapeDtypeStruct((n_ent, d_ent), entity_emb.dtype),
            jax.ShapeDtypeStruct((n_rel, d_rel), rel_emb.dtype),
        ],
    )(entity_emb, rel_emb)
    return (ent, rel)
